# initial kernel scaffold (unmeasured)
import functools

import jax
import jax.numpy as jnp
from jax import lax
from jax.experimental import pallas as pl
from jax.experimental.pallas import tpu as pltpu

N_DEV = 8
M = 2048
D = 2048
CHUNK = M // N_DEV


def kernel(partial, resid, gamma):
    partial2d = partial.reshape(M, D)
    gamma2d = gamma.reshape(1, D)

    def body(
        partial_ref,
        resid_ref,
        gamma_ref,
        out_ref,
        comm_rs,
        comm_ag,
        rs_send_sems,
        rs_recv_sems,
        ag_send_sems,
        ag_recv_sems,
    ):
        my = lax.axis_index("i")
        left = lax.rem(my - 1 + N_DEV, N_DEV)
        right = lax.rem(my + 1, N_DEV)

        barrier_sem = pltpu.get_barrier_semaphore()
        for nbr in (left, right):
            pl.semaphore_signal(
                barrier_sem,
                inc=1,
                device_id=(nbr,),
                device_id_type=pl.DeviceIdType.MESH,
            )
        pl.semaphore_wait(barrier_sem, 2)

        def rows(c):
            return pl.ds(c * CHUNK, CHUNK)

        for s in range(N_DEV - 1):
            send_c = lax.rem(my - s + 2 * N_DEV, N_DEV)
            recv_c = lax.rem(my - s - 1 + 2 * N_DEV, N_DEV)
            if s == 0:
                src = partial_ref.at[rows(send_c), :]
            else:
                src = out_ref.at[rows(send_c), :]
            rdma = pltpu.make_async_remote_copy(
                src_ref=src,
                dst_ref=comm_rs.at[s],
                send_sem=rs_send_sems.at[s],
                recv_sem=rs_recv_sems.at[s],
                device_id=(right,),
                device_id_type=pl.DeviceIdType.MESH,
            )
            rdma.start()
            rdma.wait()
            out_ref[rows(recv_c), :] = (
                comm_rs[s] + partial_ref[rows(recv_c), :]
            )

        own = lax.rem(my + 1, N_DEV)
        y = out_ref[rows(own), :] + resid_ref[rows(own), :]
        rms = jnp.sqrt(jnp.mean(y * y, axis=-1, keepdims=True) + 1e-6)
        out_ref[rows(own), :] = y / rms * gamma_ref[:, :]

        for t in range(N_DEV - 1):
            recv_c = lax.rem(my - t + 2 * N_DEV, N_DEV)
            if t == 0:
                src = out_ref.at[rows(own), :]
            else:
                src = comm_ag.at[t - 1]
            rdma = pltpu.make_async_remote_copy(
                src_ref=src,
                dst_ref=comm_ag.at[t],
                send_sem=ag_send_sems.at[t],
                recv_sem=ag_recv_sems.at[t],
                device_id=(right,),
                device_id_type=pl.DeviceIdType.MESH,
            )
            rdma.start()
            rdma.wait()
            out_ref[rows(recv_c), :] = comm_ag[t]

        @functools.partial(
            pl.run_scoped, sem=pltpu.SemaphoreType.REGULAR
        )
        def _(sem):
            for nbr in (left, right):
                pl.semaphore_signal(
                    sem,
                    inc=1,
                    device_id=(nbr,),
                    device_id_type=pl.DeviceIdType.MESH,
                )
            pl.semaphore_wait(sem, 2)

    return pl.pallas_call(
        body,
        out_shape=jax.ShapeDtypeStruct((M, D), jnp.float32),
        in_specs=[
            pl.BlockSpec(memory_space=pltpu.VMEM),
            pl.BlockSpec(memory_space=pltpu.VMEM),
            pl.BlockSpec(memory_space=pltpu.VMEM),
        ],
        out_specs=pl.BlockSpec(memory_space=pltpu.VMEM),
        scratch_shapes=[
            pltpu.VMEM((N_DEV - 1, CHUNK, D), jnp.float32),
            pltpu.VMEM((N_DEV - 1, CHUNK, D), jnp.float32),
            pltpu.SemaphoreType.DMA((N_DEV - 1,)),
            pltpu.SemaphoreType.DMA((N_DEV - 1,)),
            pltpu.SemaphoreType.DMA((N_DEV - 1,)),
            pltpu.SemaphoreType.DMA((N_DEV - 1,)),
        ],
        compiler_params=pltpu.CompilerParams(collective_id=0),
    )(partial2d, resid, gamma2d)


# baseline (device time: 367087 ns/iter reference)
import functools

import jax
import jax.numpy as jnp
from jax import lax
from jax.experimental import pallas as pl
from jax.experimental.pallas import tpu as pltpu

N_DEV = 8
M = 2048
D = 2048
CHUNK = M // N_DEV


def kernel(partial, resid, gamma):
    partial2d = partial.reshape(M, D)
    gamma2d = gamma.reshape(1, D)

    def body(
        partial_ref,
        resid_ref,
        gamma_ref,
        out_ref,
        comm_rs,
        comm_ag,
        stage_send,
        stage_acc,
        resid_stage,
        rs_send_sems,
        rs_recv_sems,
        ag_send_sems,
        ag_recv_sems,
        cp_send_sem,
        cp_acc_sem,
        cp_resid_sem,
    ):
        my = lax.axis_index("i")
        left = lax.rem(my - 1 + N_DEV, N_DEV)
        right = lax.rem(my + 1, N_DEV)
        own = lax.rem(my + 1, N_DEV)

        def rows(c):
            return pl.ds(c * CHUNK, CHUNK)

        cp_send = pltpu.make_async_copy(
            partial_ref.at[rows(my), :], stage_send, cp_send_sem
        )
        cp_send.start()
        cp_resid = pltpu.make_async_copy(
            resid_ref.at[rows(own), :], resid_stage, cp_resid_sem
        )
        cp_resid.start()

        barrier_sem = pltpu.get_barrier_semaphore()
        for nbr in (left, right):
            pl.semaphore_signal(
                barrier_sem,
                inc=1,
                device_id=(nbr,),
                device_id_type=pl.DeviceIdType.MESH,
            )
        pl.semaphore_wait(barrier_sem, 2)

        cp_send.wait()

        for s in range(N_DEV - 1):
            send_c = lax.rem(my - s + 2 * N_DEV, N_DEV)
            recv_c = lax.rem(my - s - 1 + 2 * N_DEV, N_DEV)
            if s == 0:
                src = stage_send
            else:
                src = out_ref.at[rows(send_c), :]
            rdma = pltpu.make_async_remote_copy(
                src_ref=src,
                dst_ref=comm_rs.at[s],
                send_sem=rs_send_sems.at[s],
                recv_sem=rs_recv_sems.at[s],
                device_id=(right,),
                device_id_type=pl.DeviceIdType.MESH,
            )
            rdma.start()
            cp_acc = pltpu.make_async_copy(
                partial_ref.at[rows(recv_c), :], stage_acc, cp_acc_sem
            )
            cp_acc.start()
            cp_acc.wait()
            rdma.wait()
            out_ref[rows(recv_c), :] = comm_rs[s] + stage_acc[:, :]

        cp_resid.wait()
        y = out_ref[rows(own), :] + resid_stage[:, :]
        rms = jnp.sqrt(jnp.mean(y * y, axis=-1, keepdims=True) + 1e-6)
        out_ref[rows(own), :] = y / rms * gamma_ref[:, :]

        for t in range(N_DEV - 1):
            recv_c = lax.rem(my - t + 2 * N_DEV, N_DEV)
            if t == 0:
                src = out_ref.at[rows(own), :]
            else:
                src = comm_ag.at[t - 1]
            rdma = pltpu.make_async_remote_copy(
                src_ref=src,
                dst_ref=comm_ag.at[t],
                send_sem=ag_send_sems.at[t],
                recv_sem=ag_recv_sems.at[t],
                device_id=(right,),
                device_id_type=pl.DeviceIdType.MESH,
            )
            rdma.start()
            rdma.wait()
            out_ref[rows(recv_c), :] = comm_ag[t]

        @functools.partial(
            pl.run_scoped, sem=pltpu.SemaphoreType.REGULAR
        )
        def _(sem):
            for nbr in (left, right):
                pl.semaphore_signal(
                    sem,
                    inc=1,
                    device_id=(nbr,),
                    device_id_type=pl.DeviceIdType.MESH,
                )
            pl.semaphore_wait(sem, 2)

    return pl.pallas_call(
        body,
        out_shape=jax.ShapeDtypeStruct((M, D), jnp.float32),
        in_specs=[
            pl.BlockSpec(memory_space=pl.ANY),
            pl.BlockSpec(memory_space=pl.ANY),
            pl.BlockSpec(memory_space=pltpu.VMEM),
        ],
        out_specs=pl.BlockSpec(memory_space=pltpu.VMEM),
        scratch_shapes=[
            pltpu.VMEM((N_DEV - 1, CHUNK, D), jnp.float32),
            pltpu.VMEM((N_DEV - 1, CHUNK, D), jnp.float32),
            pltpu.VMEM((CHUNK, D), jnp.float32),
            pltpu.VMEM((CHUNK, D), jnp.float32),
            pltpu.VMEM((CHUNK, D), jnp.float32),
            pltpu.SemaphoreType.DMA((N_DEV - 1,)),
            pltpu.SemaphoreType.DMA((N_DEV - 1,)),
            pltpu.SemaphoreType.DMA((N_DEV - 1,)),
            pltpu.SemaphoreType.DMA((N_DEV - 1,)),
            pltpu.SemaphoreType.DMA,
            pltpu.SemaphoreType.DMA,
            pltpu.SemaphoreType.DMA,
        ],
        compiler_params=pltpu.CompilerParams(
            collective_id=0,
            vmem_limit_bytes=60 * 1024 * 1024,
        ),
    )(partial2d, resid, gamma2d)


# device time: 214828 ns/iter; 1.7087x vs baseline; 1.7087x over previous
import functools

import jax
import jax.numpy as jnp
from jax import lax
from jax.experimental import pallas as pl
from jax.experimental.pallas import tpu as pltpu

N_DEV = 8
M = 2048
D = 2048
CHUNK = M // N_DEV
HALF = CHUNK // 2


def kernel(partial, resid, gamma):
    partial2d = partial.reshape(M, D)
    gamma2d = gamma.reshape(1, D)

    def body(
        partial_ref,
        resid_ref,
        gamma_ref,
        out_ref,
        comm_rs,
        comm_ag,
        stage_send,
        stage_acc,
        resid_stage,
        rs_send_sems,
        rs_recv_sems,
        ag_send_sems,
        ag_recv_sems,
        cp_send_sem,
        cp_acc_sems,
        cp_resid_sems,
    ):
        my = lax.axis_index("i")
        left = lax.rem(my - 1 + N_DEV, N_DEV)
        right = lax.rem(my + 1, N_DEV)
        peer = (right, left)
        own = (right, left)

        def mod(c):
            return lax.rem(c + 2 * N_DEV, N_DEV)

        def hrows(c, d):
            return pl.ds(c * CHUNK + d * HALF, HALF)

        cp_send = pltpu.make_async_copy(
            partial_ref.at[pl.ds(my * CHUNK, CHUNK), :],
            stage_send,
            cp_send_sem,
        )
        cp_send.start()
        cp_resid = []
        for d in range(2):
            cp = pltpu.make_async_copy(
                resid_ref.at[hrows(own[d], d), :],
                resid_stage.at[d],
                cp_resid_sems.at[d],
            )
            cp.start()
            cp_resid.append(cp)

        barrier_sem = pltpu.get_barrier_semaphore()
        for nbr in (left, right):
            pl.semaphore_signal(
                barrier_sem,
                inc=1,
                device_id=(nbr,),
                device_id_type=pl.DeviceIdType.MESH,
            )
        pl.semaphore_wait(barrier_sem, 2)

        cp_send.wait()

        for s in range(N_DEV - 1):
            rdmas = []
            for d in range(2):
                sgn = 1 - 2 * d
                send_c = mod(my - sgn * s)
                if s == 0:
                    src = stage_send.at[pl.ds(d * HALF, HALF), :]
                else:
                    src = out_ref.at[hrows(send_c, d), :]
                rdma = pltpu.make_async_remote_copy(
                    src_ref=src,
                    dst_ref=comm_rs.at[d, s],
                    send_sem=rs_send_sems.at[d, s],
                    recv_sem=rs_recv_sems.at[d, s],
                    device_id=(peer[d],),
                    device_id_type=pl.DeviceIdType.MESH,
                )
                rdma.start()
                rdmas.append(rdma)
            cps = []
            for d in range(2):
                sgn = 1 - 2 * d
                recv_c = mod(my - sgn * (s + 1))
                cp = pltpu.make_async_copy(
                    partial_ref.at[hrows(recv_c, d), :],
                    stage_acc.at[d],
                    cp_acc_sems.at[d],
                )
                cp.start()
                cps.append(cp)
            for d in range(2):
                cps[d].wait()
                rdmas[d].wait()
                recv_c = mod(my - (1 - 2 * d) * (s + 1))
                out_ref[hrows(recv_c, d), :] = (
                    comm_rs[d, s] + stage_acc[d, :, :]
                )

        for d in range(2):
            cp_resid[d].wait()
            rows = hrows(own[d], d)
            y = out_ref[rows, :] + resid_stage[d, :, :]
            rms = jnp.sqrt(jnp.mean(y * y, axis=-1, keepdims=True) + 1e-6)
            out_ref[rows, :] = y / rms * gamma_ref[:, :]

        for t in range(N_DEV - 1):
            rdmas = []
            for d in range(2):
                if t == 0:
                    src = out_ref.at[hrows(own[d], d), :]
                else:
                    src = comm_ag.at[d, t - 1]
                rdma = pltpu.make_async_remote_copy(
                    src_ref=src,
                    dst_ref=comm_ag.at[d, t],
                    send_sem=ag_send_sems.at[d, t],
                    recv_sem=ag_recv_sems.at[d, t],
                    device_id=(peer[d],),
                    device_id_type=pl.DeviceIdType.MESH,
                )
                rdma.start()
                rdmas.append(rdma)
            for d in range(2):
                rdmas[d].wait()
                recv_c = mod(my - (1 - 2 * d) * t)
                out_ref[hrows(recv_c, d), :] = comm_ag[d, t]

        @functools.partial(
            pl.run_scoped, sem=pltpu.SemaphoreType.REGULAR
        )
        def _(sem):
            for nbr in (left, right):
                pl.semaphore_signal(
                    sem,
                    inc=1,
                    device_id=(nbr,),
                    device_id_type=pl.DeviceIdType.MESH,
                )
            pl.semaphore_wait(sem, 2)

    return pl.pallas_call(
        body,
        out_shape=jax.ShapeDtypeStruct((M, D), jnp.float32),
        in_specs=[
            pl.BlockSpec(memory_space=pl.ANY),
            pl.BlockSpec(memory_space=pl.ANY),
            pl.BlockSpec(memory_space=pltpu.VMEM),
        ],
        out_specs=pl.BlockSpec(memory_space=pltpu.VMEM),
        scratch_shapes=[
            pltpu.VMEM((2, N_DEV - 1, HALF, D), jnp.float32),
            pltpu.VMEM((2, N_DEV - 1, HALF, D), jnp.float32),
            pltpu.VMEM((CHUNK, D), jnp.float32),
            pltpu.VMEM((2, HALF, D), jnp.float32),
            pltpu.VMEM((2, HALF, D), jnp.float32),
            pltpu.SemaphoreType.DMA((2, N_DEV - 1)),
            pltpu.SemaphoreType.DMA((2, N_DEV - 1)),
            pltpu.SemaphoreType.DMA((2, N_DEV - 1)),
            pltpu.SemaphoreType.DMA((2, N_DEV - 1)),
            pltpu.SemaphoreType.DMA,
            pltpu.SemaphoreType.DMA((2,)),
            pltpu.SemaphoreType.DMA((2,)),
        ],
        compiler_params=pltpu.CompilerParams(
            collective_id=0,
            vmem_limit_bytes=60 * 1024 * 1024,
        ),
    )(partial2d, resid, gamma2d)


# device time: 210599 ns/iter; 1.7431x vs baseline; 1.0201x over previous
import functools

import jax
import jax.numpy as jnp
from jax import lax
from jax.experimental import pallas as pl
from jax.experimental.pallas import tpu as pltpu

N_DEV = 8
M = 2048
D = 2048
CHUNK = M // N_DEV
HALF = CHUNK // 2
N_HOP = N_DEV - 1


def kernel(partial, resid, gamma):
    partial2d = partial.reshape(M, D)
    gamma2d = gamma.reshape(1, D)

    def body(
        partial_ref,
        resid_ref,
        gamma_ref,
        out_ref,
        comm_rs,
        comm_ag,
        stage_send,
        stage_acc,
        resid_stage,
        rs_send_sems,
        rs_recv_sems,
        ag_send_sems,
        ag_recv_sems,
        cp_send_sem,
        cp_acc_sems,
        cp_resid_sems,
    ):
        my = lax.axis_index("i")
        left = lax.rem(my - 1 + N_DEV, N_DEV)
        right = lax.rem(my + 1, N_DEV)
        peer = (right, left)
        own = (right, left)

        def mod(c):
            return lax.rem(c + 2 * N_DEV, N_DEV)

        def hrows(c, d):
            return pl.ds(c * CHUNK + d * HALF, HALF)

        def recv_chunk(d, s):
            return mod(my - (1 - 2 * d) * (s + 1))

        def stage_cp(d, s):
            return pltpu.make_async_copy(
                partial_ref.at[hrows(recv_chunk(d, s), d), :],
                stage_acc.at[d, s % 2],
                cp_acc_sems.at[d, s % 2],
            )

        cp_send = pltpu.make_async_copy(
            partial_ref.at[pl.ds(my * CHUNK, CHUNK), :],
            stage_send,
            cp_send_sem,
        )
        cp_send.start()
        cp_resid = []
        for d in range(2):
            cp = pltpu.make_async_copy(
                resid_ref.at[hrows(own[d], d), :],
                resid_stage.at[d],
                cp_resid_sems.at[d],
            )
            cp.start()
            cp_resid.append(cp)
        for d in range(2):
            stage_cp(d, 0).start()

        barrier_sem = pltpu.get_barrier_semaphore()
        for nbr in (left, right):
            pl.semaphore_signal(
                barrier_sem,
                inc=1,
                device_id=(nbr,),
                device_id_type=pl.DeviceIdType.MESH,
            )
        pl.semaphore_wait(barrier_sem, 2)

        cp_send.wait()

        def rs_rdma(d, s, src):
            return pltpu.make_async_remote_copy(
                src_ref=src,
                dst_ref=comm_rs.at[d, s],
                send_sem=rs_send_sems.at[d, s],
                recv_sem=rs_recv_sems.at[d, s],
                device_id=(peer[d],),
                device_id_type=pl.DeviceIdType.MESH,
            )

        rs = [[None] * N_HOP for _ in range(2)]
        for d in range(2):
            rs[d][0] = rs_rdma(d, 0, stage_send.at[pl.ds(d * HALF, HALF), :])
            rs[d][0].start()
        for s in range(N_HOP):
            if s + 1 < N_HOP:
                for d in range(2):
                    stage_cp(d, s + 1).start()
            for d in range(2):
                rs[d][s].wait_recv()
                stage_cp(d, s).wait()
                rc = recv_chunk(d, s)
                out_ref[hrows(rc, d), :] = (
                    comm_rs[d, s] + stage_acc[d, s % 2, :, :]
                )
                if s + 1 < N_HOP:
                    rs[d][s + 1] = rs_rdma(d, s + 1, out_ref.at[hrows(rc, d), :])
                    rs[d][s + 1].start()
        for d in range(2):
            for s in range(N_HOP):
                rs[d][s].wait_send()

        def ag_rdma(d, t, src):
            return pltpu.make_async_remote_copy(
                src_ref=src,
                dst_ref=comm_ag.at[d, t],
                send_sem=ag_send_sems.at[d, t],
                recv_sem=ag_recv_sems.at[d, t],
                device_id=(peer[d],),
                device_id_type=pl.DeviceIdType.MESH,
            )

        ag = [[None] * N_HOP for _ in range(2)]
        for d in range(2):
            cp_resid[d].wait()
            rows = hrows(own[d], d)
            y = out_ref[rows, :] + resid_stage[d, :, :]
            rms = jnp.sqrt(jnp.mean(y * y, axis=-1, keepdims=True) + 1e-6)
            out_ref[rows, :] = y / rms * gamma_ref[:, :]
            ag[d][0] = ag_rdma(d, 0, out_ref.at[rows, :])
            ag[d][0].start()

        for t in range(N_HOP):
            for d in range(2):
                ag[d][t].wait_recv()
                if t + 1 < N_HOP:
                    ag[d][t + 1] = ag_rdma(d, t + 1, comm_ag.at[d, t])
                    ag[d][t + 1].start()
                rc = mod(my - (1 - 2 * d) * t)
                out_ref[hrows(rc, d), :] = comm_ag[d, t]
        for d in range(2):
            for t in range(N_HOP):
                ag[d][t].wait_send()

        @functools.partial(
            pl.run_scoped, sem=pltpu.SemaphoreType.REGULAR
        )
        def _(sem):
            for nbr in (left, right):
                pl.semaphore_signal(
                    sem,
                    inc=1,
                    device_id=(nbr,),
                    device_id_type=pl.DeviceIdType.MESH,
                )
            pl.semaphore_wait(sem, 2)

    return pl.pallas_call(
        body,
        out_shape=jax.ShapeDtypeStruct((M, D), jnp.float32),
        in_specs=[
            pl.BlockSpec(memory_space=pl.ANY),
            pl.BlockSpec(memory_space=pl.ANY),
            pl.BlockSpec(memory_space=pltpu.VMEM),
        ],
        out_specs=pl.BlockSpec(memory_space=pltpu.VMEM),
        scratch_shapes=[
            pltpu.VMEM((2, N_HOP, HALF, D), jnp.float32),
            pltpu.VMEM((2, N_HOP, HALF, D), jnp.float32),
            pltpu.VMEM((CHUNK, D), jnp.float32),
            pltpu.VMEM((2, 2, HALF, D), jnp.float32),
            pltpu.VMEM((2, HALF, D), jnp.float32),
            pltpu.SemaphoreType.DMA((2, N_HOP)),
            pltpu.SemaphoreType.DMA((2, N_HOP)),
            pltpu.SemaphoreType.DMA((2, N_HOP)),
            pltpu.SemaphoreType.DMA((2, N_HOP)),
            pltpu.SemaphoreType.DMA,
            pltpu.SemaphoreType.DMA((2, 2)),
            pltpu.SemaphoreType.DMA((2,)),
        ],
        compiler_params=pltpu.CompilerParams(
            collective_id=0,
            vmem_limit_bytes=60 * 1024 * 1024,
        ),
    )(partial2d, resid, gamma2d)


# device time: 193293 ns/iter; 1.8991x vs baseline; 1.0895x over previous
import functools

import jax
import jax.numpy as jnp
from jax import lax
from jax.experimental import pallas as pl
from jax.experimental.pallas import tpu as pltpu

N_DEV = 8
M = 2048
D = 2048
CHUNK = M // N_DEV
HALF = CHUNK // 2
Q = 2
SUB = HALF // Q
N_LANE = 2 * Q
N_HOP = N_DEV - 1


def kernel(partial, resid, gamma):
    partial2d = partial.reshape(M, D)
    gamma2d = gamma.reshape(1, D)

    def body(
        partial_ref,
        resid_ref,
        gamma_ref,
        out_ref,
        comm_rs,
        comm_ag,
        stage_send,
        stage_acc,
        resid_stage,
        rs_send_sems,
        rs_recv_sems,
        ag_send_sems,
        ag_recv_sems,
        cp_send_sem,
        cp_acc_sems,
        cp_resid_sems,
    ):
        my = lax.axis_index("i")
        left = lax.rem(my - 1 + N_DEV, N_DEV)
        right = lax.rem(my + 1, N_DEV)
        lane_dir = [l // Q for l in range(N_LANE)]
        peer = [(right, left)[d] for d in lane_dir]
        own = [(right, left)[d] for d in lane_dir]

        def mod(c):
            return lax.rem(c + 2 * N_DEV, N_DEV)

        def lrows(c, l):
            off = lane_dir[l] * HALF + (l % Q) * SUB
            return pl.ds(c * CHUNK + off, SUB)

        def recv_chunk(l, s):
            return mod(my - (1 - 2 * lane_dir[l]) * (s + 1))

        def stage_cp(l, s):
            return pltpu.make_async_copy(
                partial_ref.at[lrows(recv_chunk(l, s), l), :],
                stage_acc.at[l, s % 2],
                cp_acc_sems.at[l, s % 2],
            )

        cp_send = pltpu.make_async_copy(
            partial_ref.at[pl.ds(my * CHUNK, CHUNK), :],
            stage_send,
            cp_send_sem,
        )
        cp_send.start()
        cp_resid = []
        for l in range(N_LANE):
            cp = pltpu.make_async_copy(
                resid_ref.at[lrows(own[l], l), :],
                resid_stage.at[l],
                cp_resid_sems.at[l],
            )
            cp.start()
            cp_resid.append(cp)
        for l in range(N_LANE):
            stage_cp(l, 0).start()

        barrier_sem = pltpu.get_barrier_semaphore()
        for nbr in (left, right):
            pl.semaphore_signal(
                barrier_sem,
                inc=1,
                device_id=(nbr,),
                device_id_type=pl.DeviceIdType.MESH,
            )
        pl.semaphore_wait(barrier_sem, 2)

        cp_send.wait()

        def rs_rdma(l, s, src):
            return pltpu.make_async_remote_copy(
                src_ref=src,
                dst_ref=comm_rs.at[l, s],
                send_sem=rs_send_sems.at[l, s],
                recv_sem=rs_recv_sems.at[l, s],
                device_id=(peer[l],),
                device_id_type=pl.DeviceIdType.MESH,
            )

        rs = [[None] * N_HOP for _ in range(N_LANE)]
        for l in range(N_LANE):
            off = lane_dir[l] * HALF + (l % Q) * SUB
            rs[l][0] = rs_rdma(l, 0, stage_send.at[pl.ds(off, SUB), :])
            rs[l][0].start()
        for s in range(N_HOP):
            if s + 1 < N_HOP:
                for l in range(N_LANE):
                    stage_cp(l, s + 1).start()
            for l in range(N_LANE):
                rs[l][s].wait_recv()
                stage_cp(l, s).wait()
                rc = recv_chunk(l, s)
                out_ref[lrows(rc, l), :] = (
                    comm_rs[l, s] + stage_acc[l, s % 2, :, :]
                )
                if s + 1 < N_HOP:
                    rs[l][s + 1] = rs_rdma(l, s + 1, out_ref.at[lrows(rc, l), :])
                    rs[l][s + 1].start()
        for l in range(N_LANE):
            for s in range(N_HOP):
                rs[l][s].wait_send()

        def ag_rdma(l, t, src):
            return pltpu.make_async_remote_copy(
                src_ref=src,
                dst_ref=comm_ag.at[l, t],
                send_sem=ag_send_sems.at[l, t],
                recv_sem=ag_recv_sems.at[l, t],
                device_id=(peer[l],),
                device_id_type=pl.DeviceIdType.MESH,
            )

        ag = [[None] * N_HOP for _ in range(N_LANE)]
        for l in range(N_LANE):
            cp_resid[l].wait()
            rows = lrows(own[l], l)
            y = out_ref[rows, :] + resid_stage[l, :, :]
            rms = jnp.sqrt(jnp.mean(y * y, axis=-1, keepdims=True) + 1e-6)
            out_ref[rows, :] = y / rms * gamma_ref[:, :]
            ag[l][0] = ag_rdma(l, 0, out_ref.at[rows, :])
            ag[l][0].start()

        for t in range(N_HOP):
            for l in range(N_LANE):
                ag[l][t].wait_recv()
                if t + 1 < N_HOP:
                    ag[l][t + 1] = ag_rdma(l, t + 1, comm_ag.at[l, t])
                    ag[l][t + 1].start()
                rc = mod(my - (1 - 2 * lane_dir[l]) * t)
                out_ref[lrows(rc, l), :] = comm_ag[l, t]
        for l in range(N_LANE):
            for t in range(N_HOP):
                ag[l][t].wait_send()

        @functools.partial(
            pl.run_scoped, sem=pltpu.SemaphoreType.REGULAR
        )
        def _(sem):
            for nbr in (left, right):
                pl.semaphore_signal(
                    sem,
                    inc=1,
                    device_id=(nbr,),
                    device_id_type=pl.DeviceIdType.MESH,
                )
            pl.semaphore_wait(sem, 2)

    return pl.pallas_call(
        body,
        out_shape=jax.ShapeDtypeStruct((M, D), jnp.float32),
        in_specs=[
            pl.BlockSpec(memory_space=pl.ANY),
            pl.BlockSpec(memory_space=pl.ANY),
            pl.BlockSpec(memory_space=pltpu.VMEM),
        ],
        out_specs=pl.BlockSpec(memory_space=pltpu.VMEM),
        scratch_shapes=[
            pltpu.VMEM((N_LANE, N_HOP, SUB, D), jnp.float32),
            pltpu.VMEM((N_LANE, N_HOP, SUB, D), jnp.float32),
            pltpu.VMEM((CHUNK, D), jnp.float32),
            pltpu.VMEM((N_LANE, 2, SUB, D), jnp.float32),
            pltpu.VMEM((N_LANE, SUB, D), jnp.float32),
            pltpu.SemaphoreType.DMA((N_LANE, N_HOP)),
            pltpu.SemaphoreType.DMA((N_LANE, N_HOP)),
            pltpu.SemaphoreType.DMA((N_LANE, N_HOP)),
            pltpu.SemaphoreType.DMA((N_LANE, N_HOP)),
            pltpu.SemaphoreType.DMA,
            pltpu.SemaphoreType.DMA((N_LANE, 2)),
            pltpu.SemaphoreType.DMA((N_LANE,)),
        ],
        compiler_params=pltpu.CompilerParams(
            collective_id=0,
            vmem_limit_bytes=60 * 1024 * 1024,
        ),
    )(partial2d, resid, gamma2d)
